# baseline (speedup 1.0000x reference)
import functools

import jax
import jax.numpy as jnp
from jax import lax
from jax.experimental import pallas as pl
from jax.experimental.pallas import tpu as pltpu
from jax.experimental.pallas import tpu_sc as plsc

_NC = 2
_NS = 16
_NW = _NC * _NS

_B = 4096
_BPW = _B // _NW


def _make_sc_kernel():
    @functools.partial(
        pl.kernel,
        mesh=plsc.VectorSubcoreMesh(core_axis_name="c", subcore_axis_name="s"),
        out_type=[
            jax.ShapeDtypeStruct((_B,), jnp.int32),
            jax.ShapeDtypeStruct((_B,), jnp.int32),
        ],
        scratch_types=[
            pltpu.VMEM_SHARED((_B,), jnp.int32),
            pltpu.VMEM((_BPW,), jnp.int32),
            pltpu.VMEM((_BPW,), jnp.int32),
            pltpu.VMEM((_BPW,), jnp.int32),
            pltpu.VMEM((_BPW,), jnp.int32),
            pltpu.VMEM((_BPW,), jnp.int32),
            pltpu.VMEM((_BPW,), jnp.int32),
            pltpu.VMEM((_BPW,), jnp.int32),
            pltpu.SemaphoreType.DMA,
            pltpu.SemaphoreType.DMA,
        ],
    )
    def k(ptr_hbm, flat_hbm, out_lo, out_hi,
          inv_s, idx_v, pos_v, chosen_v, ilo_v, ihi_v, lo_v, hi_v,
          sem0, sem1):
        wid = lax.axis_index("c") * _NS + lax.axis_index("s")
        base = wid * _BPW
        pltpu.sync_copy(ptr_hbm.at[pl.ds(base, _BPW)], idx_v)
        lane = lax.iota(jnp.int32, 16)
        for j in range(_BPW // 16):
            pos_v[pl.ds(j * 16, 16)] = base + j * 16 + lane
        pltpu.async_copy(pos_v, inv_s.at[idx_v], sem0).wait()
        plsc.subcore_barrier()
        pltpu.async_copy(inv_s.at[idx_v], chosen_v, sem0).wait()
        for j in range(_BPW // 16):
            c = chosen_v[pl.ds(j * 16, 16)]
            c2 = c + c
            ilo_v[pl.ds(j * 16, 16)] = c2
            ihi_v[pl.ds(j * 16, 16)] = c2 + 1
        lo_cp = pltpu.async_copy(flat_hbm.at[ilo_v], lo_v, sem0)
        hi_cp = pltpu.async_copy(flat_hbm.at[ihi_v], hi_v, sem1)
        lo_cp.wait()
        hi_cp.wait()
        pltpu.sync_copy(lo_v, out_lo.at[pl.ds(base, _BPW)])
        pltpu.sync_copy(hi_v, out_hi.at[pl.ds(base, _BPW)])

    return k


_sc_kernel = _make_sc_kernel()


def kernel(memory_addresses, pointer_ids):
    flat = lax.bitcast_convert_type(memory_addresses, jnp.int32).reshape(-1)
    ptr = pointer_ids.astype(jnp.int32)
    out_lo, out_hi = _sc_kernel(ptr, flat)
    pair = jnp.stack([out_lo, out_hi], axis=-1)
    return lax.bitcast_convert_type(pair, jnp.int64)


# speedup vs baseline: 2.5010x; 2.5010x over previous
import functools

import jax
import jax.numpy as jnp
from jax import lax
from jax.experimental import pallas as pl
from jax.experimental.pallas import tpu as pltpu
from jax.experimental.pallas import tpu_sc as plsc

_NC = 2
_NS = 16
_NW = _NC * _NS

_B = 4096
_BPW = _B // _NW


def _make_sc_kernel():
    @functools.partial(
        pl.kernel,
        mesh=plsc.VectorSubcoreMesh(core_axis_name="c", subcore_axis_name="s"),
        out_type=[
            jax.ShapeDtypeStruct((_B,), jnp.int32),
            jax.ShapeDtypeStruct((_B,), jnp.int32),
        ],
        scratch_types=[
            pltpu.VMEM_SHARED((_B,), jnp.int32),
            pltpu.VMEM((_BPW,), jnp.int32),
            pltpu.VMEM((_BPW,), jnp.int32),
            pltpu.VMEM((_BPW,), jnp.int32),
            pltpu.VMEM((_BPW,), jnp.int32),
            pltpu.VMEM((_BPW,), jnp.int32),
            pltpu.SemaphoreType.DMA,
            pltpu.SemaphoreType.DMA,
        ],
    )
    def k(ptr_hbm, lo_hbm, hi_hbm, out_lo, out_hi,
          inv_s, idx_v, pos_v, chosen_v, lo_v, hi_v, sem0, sem1):
        wid = lax.axis_index("c") * _NS + lax.axis_index("s")
        base = wid * _BPW
        pltpu.sync_copy(ptr_hbm.at[pl.ds(base, _BPW)], idx_v)
        lane = lax.iota(jnp.int32, 16)
        for j in range(_BPW // 16):
            pos_v[pl.ds(j * 16, 16)] = base + j * 16 + lane
        pltpu.async_copy(pos_v, inv_s.at[idx_v], sem0).wait()
        plsc.subcore_barrier()
        pltpu.async_copy(inv_s.at[idx_v], chosen_v, sem0).wait()
        lo_cp = pltpu.async_copy(lo_hbm.at[chosen_v], lo_v, sem0)
        hi_cp = pltpu.async_copy(hi_hbm.at[chosen_v], hi_v, sem1)
        lo_cp.wait()
        hi_cp.wait()
        pltpu.sync_copy(lo_v, out_lo.at[pl.ds(base, _BPW)])
        pltpu.sync_copy(hi_v, out_hi.at[pl.ds(base, _BPW)])

    return k


_sc_kernel = _make_sc_kernel()


def kernel(memory_addresses, pointer_ids):
    parts = lax.bitcast_convert_type(memory_addresses, jnp.int32)
    lo = parts[..., 0]
    hi = parts[..., 1]
    ptr = pointer_ids.astype(jnp.int32)
    out_lo, out_hi = _sc_kernel(ptr, lo, hi)
    pair = jnp.stack([out_lo, out_hi], axis=-1)
    return lax.bitcast_convert_type(pair, jnp.int64)


# speedup vs baseline: 2.5056x; 1.0018x over previous
import functools

import jax
import jax.numpy as jnp
from jax import lax
from jax.experimental import pallas as pl
from jax.experimental.pallas import tpu as pltpu
from jax.experimental.pallas import tpu_sc as plsc

_NC = 2
_NS = 16
_NW = _NC * _NS

_B = 4096
_BPW = _B // _NW


def _make_sc_kernel():
    @functools.partial(
        pl.kernel,
        mesh=plsc.VectorSubcoreMesh(core_axis_name="c", subcore_axis_name="s"),
        out_type=[
            jax.ShapeDtypeStruct((_B,), jnp.int32),
            jax.ShapeDtypeStruct((_B,), jnp.int32),
        ],
        scratch_types=[
            pltpu.VMEM_SHARED((_B,), jnp.int32),
            pltpu.VMEM((_BPW,), jnp.int32),
            pltpu.VMEM((_BPW,), jnp.int32),
            pltpu.VMEM((_BPW,), jnp.int32),
            pltpu.VMEM((_BPW,), jnp.int32),
            pltpu.VMEM((_BPW,), jnp.int32),
            pltpu.SemaphoreType.DMA,
            pltpu.SemaphoreType.DMA,
        ],
    )
    def k(ptr_hbm, lo_hbm, hi_hbm, out_lo, out_hi,
          inv_s, idx_v, pos_v, chosen_v, lo_v, hi_v, sem0, sem1):
        wid = lax.axis_index("c") * _NS + lax.axis_index("s")
        base = wid * _BPW
        pltpu.sync_copy(ptr_hbm.at[pl.ds(base, _BPW)], idx_v)
        lane = lax.iota(jnp.int32, 16)
        for j in range(_BPW // 16):
            pos_v[pl.ds(j * 16, 16)] = base + j * 16 + lane
        pltpu.async_copy(pos_v, inv_s.at[idx_v], sem0).wait()
        plsc.subcore_barrier()
        pltpu.async_copy(inv_s.at[idx_v], chosen_v, sem0).wait()
        lo_cp = pltpu.async_copy(lo_hbm.at[chosen_v], lo_v, sem0)
        hi_cp = pltpu.async_copy(hi_hbm.at[chosen_v], hi_v, sem1)
        lo_cp.wait()
        hi_cp.wait()
        pltpu.sync_copy(lo_v, out_lo.at[pl.ds(base, _BPW)])
        pltpu.sync_copy(hi_v, out_hi.at[pl.ds(base, _BPW)])

    return k


_sc_kernel = _make_sc_kernel()


def kernel(memory_addresses, pointer_ids):
    lo = memory_addresses.astype(jnp.int32)
    hi = (memory_addresses >> 32).astype(jnp.int32)
    ptr = pointer_ids.astype(jnp.int32)
    out_lo, out_hi = _sc_kernel(ptr, lo, hi)
    pair = jnp.stack([out_lo, out_hi], axis=-1)
    return lax.bitcast_convert_type(pair, jnp.int64)


# speedup vs baseline: 2.5115x; 1.0024x over previous
import functools

import jax
import jax.numpy as jnp
from jax import lax
from jax.experimental import pallas as pl
from jax.experimental.pallas import tpu as pltpu
from jax.experimental.pallas import tpu_sc as plsc

_NC = 2
_NS = 16
_NW = _NC * _NS

_B = 4096
_BPW = _B // _NW


def _make_sc_kernel():
    @functools.partial(
        pl.kernel,
        mesh=plsc.VectorSubcoreMesh(core_axis_name="c", subcore_axis_name="s"),
        out_type=[
            jax.ShapeDtypeStruct((_B,), jnp.int32),
            jax.ShapeDtypeStruct((_B,), jnp.int32),
        ],
        scratch_types=[
            pltpu.VMEM_SHARED((_B,), jnp.int32),
            pltpu.VMEM((_BPW,), jnp.int32),
            pltpu.VMEM((_BPW,), jnp.int32),
            pltpu.VMEM((_BPW,), jnp.int32),
            pltpu.VMEM((_BPW,), jnp.int32),
            pltpu.VMEM((_BPW,), jnp.int32),
            pltpu.SemaphoreType.DMA,
            pltpu.SemaphoreType.DMA,
        ],
    )
    def k(ptr_hbm, lo_hbm, hi_hbm, out_lo, out_hi,
          inv_s, idx_v, pos_v, chosen_v, lo_v, hi_v, sem0, sem1):
        wid = lax.axis_index("c") * _NS + lax.axis_index("s")
        base = wid * _BPW
        ptr_cp = pltpu.async_copy(ptr_hbm.at[pl.ds(base, _BPW)], idx_v, sem1)
        lane = lax.iota(jnp.int32, 16)
        for j in range(_BPW // 16):
            pos_v[pl.ds(j * 16, 16)] = base + j * 16 + lane
        ptr_cp.wait()
        pltpu.async_copy(pos_v, inv_s.at[idx_v], sem0).wait()
        plsc.subcore_barrier()
        pltpu.async_copy(inv_s.at[idx_v], chosen_v, sem0).wait()
        lo_cp = pltpu.async_copy(lo_hbm.at[chosen_v], lo_v, sem0)
        hi_cp = pltpu.async_copy(hi_hbm.at[chosen_v], hi_v, sem1)
        lo_cp.wait()
        hi_cp.wait()
        pltpu.sync_copy(lo_v, out_lo.at[pl.ds(base, _BPW)])
        pltpu.sync_copy(hi_v, out_hi.at[pl.ds(base, _BPW)])

    return k


_sc_kernel = _make_sc_kernel()


def kernel(memory_addresses, pointer_ids):
    lo = memory_addresses.astype(jnp.int32)
    hi = (memory_addresses >> 32).astype(jnp.int32)
    ptr = pointer_ids.astype(jnp.int32)
    out_lo, out_hi = _sc_kernel(ptr, lo, hi)
    pair = jnp.stack([out_lo, out_hi], axis=-1)
    return lax.bitcast_convert_type(pair, jnp.int64)


# speedup vs baseline: 2.6207x; 1.0435x over previous
import functools

import jax
import jax.numpy as jnp
from jax import lax
from jax.experimental import pallas as pl
from jax.experimental.pallas import tpu as pltpu
from jax.experimental.pallas import tpu_sc as plsc

_NC = 2
_NS = 16
_NW = _NC * _NS

_B = 4096
_BPW = _B // _NW


def _make_sc_kernel():
    @functools.partial(
        pl.kernel,
        mesh=plsc.VectorSubcoreMesh(core_axis_name="c", subcore_axis_name="s"),
        out_type=[
            jax.ShapeDtypeStruct((_B,), jnp.int32),
            jax.ShapeDtypeStruct((_B,), jnp.int32),
        ],
        scratch_types=[
            pltpu.VMEM_SHARED((_B,), jnp.int32),
            pltpu.VMEM_SHARED((_B,), jnp.int32),
            pltpu.VMEM((_BPW,), jnp.int32),
            pltpu.VMEM((_BPW,), jnp.int32),
            pltpu.VMEM((_BPW,), jnp.int32),
            pltpu.VMEM((_BPW,), jnp.int32),
            pltpu.VMEM((_BPW,), jnp.int32),
            pltpu.SemaphoreType.DMA,
            pltpu.SemaphoreType.DMA,
            pltpu.SemaphoreType.DMA,
        ],
    )
    def k(ptr_hbm, lo_hbm, hi_hbm, out_lo, out_hi,
          booklo_s, bookhi_s, idx_v, slo_v, shi_v, dlo_v, dhi_v,
          sem0, sem1, sem2):
        wid = lax.axis_index("c") * _NS + lax.axis_index("s")
        base = wid * _BPW
        ptr_cp = pltpu.async_copy(ptr_hbm.at[pl.ds(base, _BPW)], idx_v, sem0)
        lo_cp = pltpu.async_copy(lo_hbm.at[pl.ds(base, _BPW)], slo_v, sem1)
        hi_cp = pltpu.async_copy(hi_hbm.at[pl.ds(base, _BPW)], shi_v, sem2)
        ptr_cp.wait()
        lo_cp.wait()
        hi_cp.wait()
        blo_cp = pltpu.async_copy(slo_v, booklo_s.at[idx_v], sem1)
        bhi_cp = pltpu.async_copy(shi_v, bookhi_s.at[idx_v], sem2)
        blo_cp.wait()
        bhi_cp.wait()
        plsc.subcore_barrier()
        dlo_cp = pltpu.async_copy(booklo_s.at[idx_v], dlo_v, sem1)
        dhi_cp = pltpu.async_copy(bookhi_s.at[idx_v], dhi_v, sem2)
        dlo_cp.wait()
        dhi_cp.wait()
        pltpu.sync_copy(dlo_v, out_lo.at[pl.ds(base, _BPW)])
        pltpu.sync_copy(dhi_v, out_hi.at[pl.ds(base, _BPW)])

    return k


_sc_kernel = _make_sc_kernel()


def kernel(memory_addresses, pointer_ids):
    lo = memory_addresses.astype(jnp.int32)
    hi = (memory_addresses >> 32).astype(jnp.int32)
    ptr = pointer_ids.astype(jnp.int32)
    out_lo, out_hi = _sc_kernel(ptr, lo, hi)
    pair = jnp.stack([out_lo, out_hi], axis=-1)
    return lax.bitcast_convert_type(pair, jnp.int64)


# speedup vs baseline: 2.7388x; 1.0451x over previous
import functools

import jax
import jax.numpy as jnp
from jax import lax
from jax.experimental import pallas as pl
from jax.experimental.pallas import tpu as pltpu
from jax.experimental.pallas import tpu_sc as plsc

_NC = 2
_NS = 16
_NW = _NC * _NS

_B = 4096
_BPW = _B // _NW


def _make_sc_kernel():
    @functools.partial(
        pl.kernel,
        mesh=plsc.VectorSubcoreMesh(core_axis_name="c", subcore_axis_name="s"),
        out_type=[
            jax.ShapeDtypeStruct((_B,), jnp.int32),
            jax.ShapeDtypeStruct((_B,), jnp.int32),
        ],
        scratch_types=[
            pltpu.VMEM_SHARED((_B,), jnp.int32),
            pltpu.VMEM_SHARED((_B,), jnp.int32),
            pltpu.VMEM((_BPW,), jnp.int32),
            pltpu.VMEM((_BPW,), jnp.int32),
            pltpu.VMEM((_BPW,), jnp.int32),
            pltpu.VMEM((_BPW,), jnp.int32),
            pltpu.VMEM((_BPW,), jnp.int32),
            pltpu.SemaphoreType.DMA,
            pltpu.SemaphoreType.DMA,
            pltpu.SemaphoreType.DMA,
        ],
    )
    def k(ptr_hbm, lo_hbm, hi_hbm, out_lo, out_hi,
          booklo_s, bookhi_s, idx_v, slo_v, shi_v, dlo_v, dhi_v,
          sem0, sem1, sem2):
        wid = lax.axis_index("c") * _NS + lax.axis_index("s")
        base = wid * _BPW
        ptr_cp = pltpu.async_copy(ptr_hbm.at[pl.ds(base, _BPW)], idx_v, sem0)
        lo_cp = pltpu.async_copy(lo_hbm.at[pl.ds(base, _BPW)], slo_v, sem1)
        hi_cp = pltpu.async_copy(hi_hbm.at[pl.ds(base, _BPW)], shi_v, sem2)
        ptr_cp.wait()
        lo_cp.wait()
        hi_cp.wait()
        blo_cp = pltpu.async_copy(slo_v, booklo_s.at[idx_v], sem1)
        bhi_cp = pltpu.async_copy(shi_v, bookhi_s.at[idx_v], sem2)
        blo_cp.wait()
        bhi_cp.wait()
        plsc.subcore_barrier()
        dlo_cp = pltpu.async_copy(booklo_s.at[idx_v], dlo_v, sem1)
        dhi_cp = pltpu.async_copy(bookhi_s.at[idx_v], dhi_v, sem2)
        dlo_cp.wait()
        dhi_cp.wait()
        olo_cp = pltpu.async_copy(dlo_v, out_lo.at[pl.ds(base, _BPW)], sem1)
        ohi_cp = pltpu.async_copy(dhi_v, out_hi.at[pl.ds(base, _BPW)], sem2)
        olo_cp.wait()
        ohi_cp.wait()

    return k


_sc_kernel = _make_sc_kernel()


def kernel(memory_addresses, pointer_ids):
    reserved = memory_addresses[:_B]
    lo = reserved.astype(jnp.int32)
    hi = (reserved >> 32).astype(jnp.int32)
    ptr = pointer_ids.astype(jnp.int32)
    out_lo, out_hi = _sc_kernel(ptr, lo, hi)
    pair = jnp.stack([out_lo, out_hi], axis=-1)
    return lax.bitcast_convert_type(pair, jnp.int64)
